# Initial kernel scaffold; baseline (speedup 1.0000x reference)
import jax
import jax.numpy as jnp
from jax.experimental import pallas as pl


def kernel(w_tensor, table):
    raise NotImplementedError("write your pallas kernel here")


# speedup vs baseline: 4.2456x; 4.2456x over previous
import functools

import jax
import jax.numpy as jnp
from jax import lax
from jax.experimental import pallas as pl
from jax.experimental.pallas import tpu as pltpu
from jax.experimental.pallas import tpu_sc as plsc

NC = 2
NS = 16
NW = NC * NS
CH = 128
NBUF = 4


def _emb_body(idx_hbm, table_hbm, out_hbm, idx_v, rows_v, gsem, osem):
  n_chunks = idx_hbm.shape[1]
  per_w = n_chunks * CH
  wid = lax.axis_index("s") * NC + lax.axis_index("c")
  base = wid * per_w
  pltpu.sync_copy(idx_hbm.at[wid], idx_v)

  @pl.loop(0, n_chunks // NBUF)
  def _group(g):
    j0 = g * NBUF
    gathers = []
    for b in range(NBUF):
      gathers.append(
          pltpu.async_copy(table_hbm.at[idx_v.at[j0 + b]], rows_v.at[b], gsem)
      )
    writes = []
    for b in range(NBUF):
      gathers[b].wait()
      writes.append(
          pltpu.async_copy(
              rows_v.at[b], out_hbm.at[pl.ds(base + (j0 + b) * CH, CH)], osem
          )
      )
    for b in range(NBUF):
      writes[b].wait()


def kernel(w_tensor, table):
  B, L = w_tensor.shape
  V, D = table.shape
  tot = B * L
  per_w = tot // NW
  n_chunks = per_w // CH
  idx = w_tensor.astype(jnp.int32).reshape(NW, n_chunks, CH)

  mesh = plsc.VectorSubcoreMesh(
      core_axis_name="c", subcore_axis_name="s", num_cores=NC, num_subcores=NS
  )
  emb = functools.partial(
      pl.kernel,
      out_type=jax.ShapeDtypeStruct((tot, D), jnp.float32),
      mesh=mesh,
      scratch_types=[
          pltpu.VMEM((n_chunks, CH), jnp.int32),
          pltpu.VMEM((NBUF, CH, D), jnp.float32),
          pltpu.SemaphoreType.DMA,
          pltpu.SemaphoreType.DMA,
      ],
      compiler_params=pltpu.CompilerParams(use_tc_tiling_on_sc=False),
  )(_emb_body)
  out = emb(idx, table)
  return out.reshape(B, L, D)


# speedup vs baseline: 4.2574x; 1.0028x over previous
import functools

import jax
import jax.numpy as jnp
from jax import lax
from jax.experimental import pallas as pl
from jax.experimental.pallas import tpu as pltpu
from jax.experimental.pallas import tpu_sc as plsc

NC = 2
NS = 16
NW = NC * NS
CH = 128
NH = 4


def _emb_body(idx_hbm, table_hbm, out_hbm, idx_v, rows_v, gsem0, gsem1, osem):
  n_chunks = idx_hbm.shape[1]
  per_w = n_chunks * CH
  n_groups = n_chunks // NH
  wid = lax.axis_index("s") * NC + lax.axis_index("c")
  base = wid * per_w
  pltpu.sync_copy(idx_hbm.at[wid], idx_v)

  gsems = (gsem0, gsem1)

  def fire_group(g, half, sem):
    for b in range(NH):
      pltpu.async_copy(
          table_hbm.at[idx_v.at[g * NH + b]], rows_v.at[half * NH + b], sem
      )

  def wait_gather(half, b, sem):
    pltpu.make_async_copy(
        out_hbm.at[pl.ds(base, CH)], rows_v.at[half * NH + b], sem
    ).wait()

  def drain_writes():
    for _ in range(NH):
      pltpu.make_async_copy(
          rows_v.at[0], out_hbm.at[pl.ds(base, CH)], osem
      ).wait()

  fire_group(0, 0, gsems[0])

  @pl.loop(0, n_groups // 2)
  def _pair(p):
    for half in range(2):
      g = 2 * p + half

      @pl.when(g > 0)
      def _():
        drain_writes()

      @pl.when(g + 1 < n_groups)
      def _():
        fire_group(g + 1, 1 - half, gsems[1 - half])

      for b in range(NH):
        wait_gather(half, b, gsems[half])
        pltpu.async_copy(
            rows_v.at[half * NH + b],
            out_hbm.at[pl.ds(base + (g * NH + b) * CH, CH)],
            osem,
        )

  drain_writes()


def kernel(w_tensor, table):
  B, L = w_tensor.shape
  V, D = table.shape
  tot = B * L
  per_w = tot // NW
  n_chunks = per_w // CH
  idx = w_tensor.astype(jnp.int32).reshape(NW, n_chunks, CH)

  mesh = plsc.VectorSubcoreMesh(
      core_axis_name="c", subcore_axis_name="s", num_cores=NC, num_subcores=NS
  )
  emb = functools.partial(
      pl.kernel,
      out_type=jax.ShapeDtypeStruct((tot, D), jnp.float32),
      mesh=mesh,
      scratch_types=[
          pltpu.VMEM((n_chunks, CH), jnp.int32),
          pltpu.VMEM((2 * NH, CH, D), jnp.float32),
          pltpu.SemaphoreType.DMA,
          pltpu.SemaphoreType.DMA,
          pltpu.SemaphoreType.DMA,
      ],
      compiler_params=pltpu.CompilerParams(use_tc_tiling_on_sc=False),
  )(_emb_body)
  out = emb(idx, table)
  return out.reshape(B, L, D)


# speedup vs baseline: 4.2747x; 1.0041x over previous
import functools

import jax
import jax.numpy as jnp
from jax import lax
from jax.experimental import pallas as pl
from jax.experimental.pallas import tpu as pltpu
from jax.experimental.pallas import tpu_sc as plsc

NC = 2
NS = 16
NW = NC * NS
C0 = 104
NH = 4


def _emb_body(idx_hbm, table_hbm, out_hbm, idx_v, rows_v, gsem0, gsem1, osem):
  rows_per_w = idx_hbm.shape[0] // NW
  seq = idx_hbm.shape[1]
  sizes = (C0, seq - C0)
  n_chunks = rows_per_w * 2
  n_groups = n_chunks // NH
  wid = lax.axis_index("s") * NC + lax.axis_index("c")
  row0 = wid * rows_per_w
  pltpu.sync_copy(idx_hbm.at[pl.ds(row0, rows_per_w)], idx_v)

  gsems = (gsem0, gsem1)

  def fire_group(g, half, sem):
    for b in range(NH):
      c = g * NH + b
      r, h = c // 2, b % 2
      pltpu.async_copy(
          table_hbm.at[idx_v.at[r, pl.ds(h * C0, sizes[h])]],
          rows_v.at[half * NH + b, pl.ds(0, sizes[h])],
          sem,
      )

  def wait_gather(half, b, sem):
    h = b % 2
    pltpu.make_async_copy(
        out_hbm.at[row0, pl.ds(h * C0, sizes[h])],
        rows_v.at[half * NH + b, pl.ds(0, sizes[h])],
        sem,
    ).wait()

  def drain_writes():
    for b in range(NH):
      h = b % 2
      pltpu.make_async_copy(
          rows_v.at[0, pl.ds(0, sizes[h])],
          out_hbm.at[row0, pl.ds(h * C0, sizes[h])],
          osem,
      ).wait()

  fire_group(0, 0, gsems[0])

  @pl.loop(0, n_groups // 2)
  def _pair(p):
    for half in range(2):
      g = 2 * p + half

      @pl.when(g > 0)
      def _():
        drain_writes()

      @pl.when(g + 1 < n_groups)
      def _():
        fire_group(g + 1, 1 - half, gsems[1 - half])

      for b in range(NH):
        c = g * NH + b
        r, h = c // 2, b % 2
        wait_gather(half, b, gsems[half])
        pltpu.async_copy(
            rows_v.at[half * NH + b, pl.ds(0, sizes[h])],
            out_hbm.at[row0 + r, pl.ds(h * C0, sizes[h])],
            osem,
        )

  drain_writes()


def kernel(w_tensor, table):
  B, L = w_tensor.shape
  V, D = table.shape
  rows_per_w = B // NW

  mesh = plsc.VectorSubcoreMesh(
      core_axis_name="c", subcore_axis_name="s", num_cores=NC, num_subcores=NS
  )
  emb = functools.partial(
      pl.kernel,
      out_type=jax.ShapeDtypeStruct((B, L, D), jnp.float32),
      mesh=mesh,
      scratch_types=[
          pltpu.VMEM((rows_per_w, L), jnp.int32),
          pltpu.VMEM((2 * NH, C0, D), jnp.float32),
          pltpu.SemaphoreType.DMA,
          pltpu.SemaphoreType.DMA,
          pltpu.SemaphoreType.DMA,
      ],
      compiler_params=pltpu.CompilerParams(use_tc_tiling_on_sc=False),
  )(_emb_body)
  return emb(w_tensor.astype(jnp.int32), table)
